# baseline (speedup 1.0000x reference)
import functools

import jax
import jax.numpy as jnp
from jax import lax
from jax.experimental import pallas as pl
from jax.experimental.pallas import tpu as pltpu
import jax.experimental.pallas.tpu_sc as plsc

N = 50000
E = 800000
H = 8
C = 16
NT = 32
SC_W1 = 1000
SC_W2 = 200
R_BLK = 2000

_SC_PARAMS = pltpu.CompilerParams(
    use_tc_tiling_on_sc=False, needs_layout_passes=False)


def _io16():
    return lax.iota(jnp.int32, 16)


def _leaky(v):
    return jnp.where(v >= 0.0, v, 0.2 * v)


def _elu(v):
    return jnp.where(v > 0.0, v, jnp.exp(jnp.minimum(v, 0.0)) - 1.0)


def _tc_dense(n, f_in, r):
    nblk = n // r

    def body(x_ref, w_ref, ab_ref, ht_ref, aa_ref, aap_ref, gmax_ref):
        i = pl.program_id(0)
        h = jnp.dot(x_ref[...], w_ref[...], preferred_element_type=jnp.float32)
        for g in range(4):
            ht_ref[g] = h[:, 32 * g:32 * g + 32]
        aa_all = jnp.dot(h, ab_ref[...], preferred_element_type=jnp.float32)
        aa_ref[...] = aa_all[:, :16]
        aap_ref[...] = aa_all[:, 16:]
        bm = jnp.max(aa_all[:, :8], axis=0, keepdims=True)

        @pl.when(i == 0)
        def _():
            gmax_ref[...] = jnp.full((1, 8), -1e30, jnp.float32)

        gmax_ref[...] = jnp.maximum(gmax_ref[...], bm)

    return pl.pallas_call(
        body,
        grid=(nblk,),
        in_specs=[
            pl.BlockSpec((r, f_in), lambda i: (i, 0)),
            pl.BlockSpec((f_in, 128), lambda i: (0, 0)),
            pl.BlockSpec((128, 32), lambda i: (0, 0)),
        ],
        out_specs=[
            pl.BlockSpec((4, r, 32), lambda i: (0, i, 0)),
            pl.BlockSpec((r, 16), lambda i: (i, 0)),
            pl.BlockSpec((r, 16), lambda i: (i, 0)),
            pl.BlockSpec((1, 8), lambda i: (0, 0)),
        ],
        out_shape=[
            jax.ShapeDtypeStruct((4, n, 32), jnp.float32),
            jax.ShapeDtypeStruct((n, 16), jnp.float32),
            jax.ShapeDtypeStruct((n, 16), jnp.float32),
            jax.ShapeDtypeStruct((1, 8), jnp.float32),
        ],
    )


def _tc_combine(n, r):
    nblk = n // r

    def body(up_ref, sp_ref, aap_ref, gm_ref, ht_ref, b_ref, x_ref):
        as_ = aap_ref[:, :8]
        ad = aap_ref[:, 8:]
        g = gm_ref[...]
        eself = jnp.exp(_leaky(as_ + ad) - _leaky(ad + g))
        s_tot = sp_ref[0, :, :8] + sp_ref[1, :, :8] + eself
        rin = 1.0 / (s_tot + 1e-16)
        b = b_ref[...]
        for gi in range(4):
            es = jnp.concatenate(
                [jnp.broadcast_to(eself[:, 2 * gi:2 * gi + 1], (r, 16)),
                 jnp.broadcast_to(eself[:, 2 * gi + 1:2 * gi + 2], (r, 16))],
                axis=1)
            ri = jnp.concatenate(
                [jnp.broadcast_to(rin[:, 2 * gi:2 * gi + 1], (r, 16)),
                 jnp.broadcast_to(rin[:, 2 * gi + 1:2 * gi + 2], (r, 16))],
                axis=1)
            u = up_ref[gi, 0] + up_ref[gi, 1] + ht_ref[gi] * es
            x_ref[:, 32 * gi:32 * gi + 32] = _elu(
                u * ri + b[:, 32 * gi:32 * gi + 32])

    return pl.pallas_call(
        body,
        grid=(nblk,),
        in_specs=[
            pl.BlockSpec((4, 2, r, 32), lambda i: (0, 0, i, 0)),
            pl.BlockSpec((2, r, 16), lambda i: (0, i, 0)),
            pl.BlockSpec((r, 16), lambda i: (i, 0)),
            pl.BlockSpec((1, 8), lambda i: (0, 0)),
            pl.BlockSpec((4, r, 32), lambda i: (0, i, 0)),
            pl.BlockSpec((1, 128), lambda i: (0, 0)),
        ],
        out_specs=pl.BlockSpec((r, 128), lambda i: (i, 0)),
        out_shape=jax.ShapeDtypeStruct((n, 128), jnp.float32),
    )


def _tc_gcn_pre(n, r):
    nblk = n // r

    def body(x_ref, wg_ref, degp_ref, hs_ref, hg_ref, dinv_ref):
        hg = jnp.dot(x_ref[...], wg_ref[...], preferred_element_type=jnp.float32)
        deg = degp_ref[0] + degp_ref[1] + 1.0
        dinv = lax.rsqrt(deg)
        hg_ref[...] = hg
        dinv_ref[...] = dinv
        hs_ref[...] = hg * dinv

    return pl.pallas_call(
        body,
        grid=(nblk,),
        in_specs=[
            pl.BlockSpec((r, 128), lambda i: (i, 0)),
            pl.BlockSpec((128, 16), lambda i: (0, 0)),
            pl.BlockSpec((2, r, 16), lambda i: (0, i, 0)),
        ],
        out_specs=[
            pl.BlockSpec((r, 16), lambda i: (i, 0)),
            pl.BlockSpec((r, 16), lambda i: (i, 0)),
            pl.BlockSpec((r, 16), lambda i: (i, 0)),
        ],
        out_shape=[
            jax.ShapeDtypeStruct((n, 16), jnp.float32),
            jax.ShapeDtypeStruct((n, 16), jnp.float32),
            jax.ShapeDtypeStruct((n, 16), jnp.float32),
        ],
    )


def _tc_head(n, r):
    nblk = n // r

    def body(accp_ref, hg_ref, dinv_ref, bg_ref, wf_ref, bf_ref, y_ref):
        dinv = dinv_ref[...]
        acc = accp_ref[0] + accp_ref[1]
        out3 = _elu(dinv * acc + hg_ref[...] * dinv * dinv + bg_ref[...])
        y = jnp.sum(out3 * wf_ref[...], axis=1, keepdims=True) + bf_ref[...]
        y_ref[...] = jax.nn.sigmoid(y)

    return pl.pallas_call(
        body,
        grid=(nblk,),
        in_specs=[
            pl.BlockSpec((2, r, 16), lambda i: (0, i, 0)),
            pl.BlockSpec((r, 16), lambda i: (i, 0)),
            pl.BlockSpec((r, 16), lambda i: (i, 0)),
            pl.BlockSpec((1, 16), lambda i: (0, 0)),
            pl.BlockSpec((1, 16), lambda i: (0, 0)),
            pl.BlockSpec((1, 1), lambda i: (0, 0)),
        ],
        out_specs=pl.BlockSpec((r, 1), lambda i: (i, 0)),
        out_shape=jax.ShapeDtypeStruct((n, 1), jnp.float32),
    )


def _sc_sweep1(n, e, w):
    ept = e // NT
    nwin = ept // w
    rpt = n // 16
    mesh = plsc.VectorSubcoreMesh(core_axis_name="c", subcore_axis_name="s")

    @functools.partial(
        pl.kernel, mesh=mesh, compiler_params=_SC_PARAMS,
        out_type=[
            jax.ShapeDtypeStruct((8, e), jnp.float32),
            jax.ShapeDtypeStruct((2, n, 16), jnp.float32),
        ],
        scratch_types=[
            pltpu.VMEM((w,), jnp.int32),
            pltpu.VMEM((w,), jnp.int32),
            pltpu.VMEM((w, 16), jnp.float32),
            pltpu.VMEM((w, 16), jnp.float32),
            pltpu.VMEM((8 * w,), jnp.float32),
            pltpu.VMEM((w, 16), jnp.float32),
            pltpu.VMEM((16,), jnp.float32),
            pltpu.VMEM_SHARED((n, 16), jnp.float32),
            pltpu.SemaphoreType.DMA,
            pltpu.SemaphoreType.DMA,
        ])
    def k(aa_hbm, g_hbm, ei_hbm, z_hbm, ext_hbm, sp_hbm,
          idx_s, idx_d, rows_s, rows_d, exw, sw, gvec, s_acc, sem1, sem2):
        c = lax.axis_index("c")
        s = lax.axis_index("s")
        tid = c * 16 + s
        pltpu.sync_copy(z_hbm.at[pl.ds(s * rpt, rpt)],
                        s_acc.at[pl.ds(s * rpt, rpt)])
        pltpu.sync_copy(g_hbm, gvec)
        plsc.subcore_barrier()
        gv = gvec[...]

        def window(wi, _):
            base = tid * ept + wi * w
            pltpu.sync_copy(ei_hbm.at[0].at[pl.ds(base, w)], idx_s)
            pltpu.sync_copy(ei_hbm.at[1].at[pl.ds(base, w)], idx_d)
            cp1 = pltpu.async_copy(aa_hbm.at[idx_s], rows_s, sem1)
            cp2 = pltpu.async_copy(aa_hbm.at[idx_d], rows_d, sem2)
            cp1.wait()
            cp2.wait()

            @plsc.parallel_loop(0, w, unroll=16)
            def edge(j):
                io = _io16()
                va = rows_s[j, :]
                vd = rows_d[j, :]
                rot = lax.rev(vd, (0,))
                ex = jnp.exp(_leaky(va + rot) - _leaky(rot + gv))
                plsc.store_scatter(exw, [(io & 7) * w + j], ex, mask=io < 8)
                sw[j, :] = ex
            for hd in range(8):
                pltpu.sync_copy(exw.at[pl.ds(hd * w, w)],
                                ext_hbm.at[hd, pl.ds(base, w)])
            pltpu.sync_copy(sw, s_acc.at[idx_d], add=True)
            return 0

        lax.fori_loop(0, nwin, window, 0, unroll=False)
        plsc.subcore_barrier()
        pltpu.sync_copy(s_acc.at[pl.ds(s * rpt, rpt)],
                        sp_hbm.at[c, pl.ds(s * rpt, rpt)])

    return k


def _sc_deg(n, e, w):
    ept = e // NT
    nwin = ept // w
    rpt = n // 16
    mesh = plsc.VectorSubcoreMesh(core_axis_name="c", subcore_axis_name="s")

    @functools.partial(
        pl.kernel, mesh=mesh, compiler_params=_SC_PARAMS,
        out_type=jax.ShapeDtypeStruct((2, n, 16), jnp.float32),
        scratch_types=[
            pltpu.VMEM((w,), jnp.int32),
            pltpu.VMEM((w,), jnp.float32),
            pltpu.VMEM((w, 16), jnp.float32),
            pltpu.VMEM_SHARED((n, 16), jnp.float32),
        ])
    def k(ei_hbm, wgt_hbm, z_hbm, degp_hbm, idx_d, wbuf, wrows, deg_acc):
        c = lax.axis_index("c")
        s = lax.axis_index("s")
        tid = c * 16 + s
        pltpu.sync_copy(z_hbm.at[pl.ds(s * rpt, rpt)],
                        deg_acc.at[pl.ds(s * rpt, rpt)])
        plsc.subcore_barrier()

        def window(wi, _):
            base = tid * ept + wi * w
            pltpu.sync_copy(ei_hbm.at[1].at[pl.ds(base, w)], idx_d)
            pltpu.sync_copy(wgt_hbm.at[pl.ds(base, w)], wbuf)

            @plsc.parallel_loop(0, w, unroll=8)
            def edge(j):
                io = _io16()
                wrows[j, :] = plsc.load_gather(wbuf, [io * 0 + j])
            pltpu.sync_copy(wrows, deg_acc.at[idx_d], add=True)
            return 0

        lax.fori_loop(0, nwin, window, 0, unroll=False)
        plsc.subcore_barrier()
        pltpu.sync_copy(deg_acc.at[pl.ds(s * rpt, rpt)],
                        degp_hbm.at[c, pl.ds(s * rpt, rpt)])

    return k


def _sweep_pipeline(nwin, issue_a, wait_a, issue_d, wait_d, issue_g,
                    wait_g, compute, issue_s, wait_s):
    issue_a(0, 0)
    issue_d(0, 0)
    issue_a(1, 1)
    wait_a(0)
    issue_g(0, 0)
    wait_g(0)
    compute(0, 0)
    wait_d(0)
    issue_s(0, 0)
    wait_a(1)
    issue_d(1, 1)
    issue_g(1, 1)
    issue_a(2, 0)

    def pair(m, _):
        for i in (0, 1):
            b = (1 + i) & 1
            bn = 1 - b
            k = 1 + 2 * m + i
            wait_g(b)
            compute(k, b)
            wait_d(b)
            issue_s(k, b)

            @pl.when(k + 1 < nwin)
            def _():
                wait_a(bn)
                wait_s(bn)
                issue_d(k + 1, bn)
                issue_g(k + 1, bn)

            @pl.when(k + 2 < nwin)
            def _():
                issue_a(k + 2, b)
        return 0

    lax.fori_loop(0, (nwin - 1) // 2, pair, 0, unroll=False)
    wait_s(0)
    wait_s(1)


def _sc_sweep2(n, e, w):
    ept = e // NT
    nwin = ept // w
    rpt = n // 16
    mesh = plsc.VectorSubcoreMesh(core_axis_name="c", subcore_axis_name="s")

    @functools.partial(
        pl.kernel, mesh=mesh, compiler_params=_SC_PARAMS,
        out_type=jax.ShapeDtypeStruct((4, 2, n, 32), jnp.float32),
        scratch_types=[
            pltpu.VMEM((2, w), jnp.int32),
            pltpu.VMEM((2, w), jnp.int32),
            pltpu.VMEM((2, w, 32), jnp.float32),
            pltpu.VMEM((2, 2, w), jnp.float32),
            pltpu.VMEM_SHARED((n, 32), jnp.float32),
            pltpu.SemaphoreType.DMA,
            pltpu.SemaphoreType.DMA,
            pltpu.SemaphoreType.DMA,
            pltpu.SemaphoreType.DMA,
            pltpu.SemaphoreType.DMA,
            pltpu.SemaphoreType.DMA,
            pltpu.SemaphoreType.DMA,
            pltpu.SemaphoreType.DMA,
        ])
    def k(ht_hbm, ei_hbm, ext_hbm, z_hbm, out_hbm,
          idx_s, idx_ds, rows, ex2, acc, sa0, sa1, sd0, sd1, sg0, sg1,
          ss0, ss1):
        c = lax.axis_index("c")
        s = lax.axis_index("s")
        tid = c * 16 + s
        t0 = tid * ept
        semA = (sa0, sa1)
        semD = (sd0, sd1)
        semG = (sg0, sg1)
        semS = (ss0, ss1)

        for g in range(4):
            pltpu.sync_copy(z_hbm.at[pl.ds(s * rpt, rpt)],
                            acc.at[pl.ds(s * rpt, rpt)])
            plsc.subcore_barrier()

            def issue_a(k_, b, g=g):
                base = t0 + k_ * w
                pltpu.async_copy(ei_hbm.at[0].at[pl.ds(base, w)],
                                 idx_s.at[b], semA[b])
                pltpu.async_copy(ext_hbm.at[pl.ds(2 * g, 2), pl.ds(base, w)],
                                 ex2.at[b], semA[b])

            def wait_a(b, g=g):
                pltpu.make_async_copy(ei_hbm.at[0].at[pl.ds(0, w)],
                                      idx_s.at[b], semA[b]).wait()
                pltpu.make_async_copy(
                    ext_hbm.at[pl.ds(2 * g, 2), pl.ds(0, w)], ex2.at[b],
                    semA[b]).wait()

            def issue_d(k_, b):
                base = t0 + k_ * w
                pltpu.async_copy(ei_hbm.at[1].at[pl.ds(base, w)],
                                 idx_ds.at[b], semD[b])

            def wait_d(b):
                pltpu.make_async_copy(ei_hbm.at[1].at[pl.ds(0, w)],
                                      idx_ds.at[b], semD[b]).wait()

            def issue_g(k_, b, g=g):
                pltpu.async_copy(ht_hbm.at[g].at[idx_s.at[b]], rows.at[b],
                                 semG[b])

            def wait_g(b, g=g):
                pltpu.make_async_copy(ht_hbm.at[g].at[idx_s.at[b]],
                                      rows.at[b], semG[b]).wait()

            def compute(k_, b):
                @plsc.parallel_loop(0, w, unroll=16)
                def edge(j):
                    io = _io16()
                    b0 = plsc.load_gather(ex2.at[b, 0], [io * 0 + j])
                    b1 = plsc.load_gather(ex2.at[b, 1], [io * 0 + j])
                    rows[b, j, pl.ds(0, 16)] = rows[b, j, pl.ds(0, 16)] * b0
                    rows[b, j, pl.ds(16, 16)] = rows[b, j, pl.ds(16, 16)] * b1

            def issue_s(k_, b):
                pltpu.async_copy(rows.at[b], acc.at[idx_ds.at[b]], semS[b],
                                 add=True)

            def wait_s(b):
                pltpu.make_async_copy(rows.at[b], acc.at[idx_ds.at[b]],
                                      semS[b]).wait()

            _sweep_pipeline(nwin, issue_a, wait_a, issue_d, wait_d, issue_g,
                            wait_g, compute, issue_s, wait_s)
            plsc.subcore_barrier()
            pltpu.sync_copy(acc.at[pl.ds(s * rpt, rpt)],
                            out_hbm.at[g, c, pl.ds(s * rpt, rpt)])
            plsc.subcore_barrier()

    return k


def _sc_sweep3(n, e, w):
    ept = e // NT
    nwin = ept // w
    rpt = n // 16
    mesh = plsc.VectorSubcoreMesh(core_axis_name="c", subcore_axis_name="s")

    @functools.partial(
        pl.kernel, mesh=mesh, compiler_params=_SC_PARAMS,
        out_type=jax.ShapeDtypeStruct((2, n, 16), jnp.float32),
        scratch_types=[
            pltpu.VMEM((w,), jnp.int32),
            pltpu.VMEM((w,), jnp.int32),
            pltpu.VMEM((w,), jnp.float32),
            pltpu.VMEM((w, 16), jnp.float32),
            pltpu.VMEM_SHARED((n, 16), jnp.float32),
            pltpu.SemaphoreType.DMA,
        ])
    def k(hs_hbm, ei_hbm, wgt_hbm, z_hbm, out_hbm,
          idx_s, idx_d, wbuf, rows, acc, sem):
        c = lax.axis_index("c")
        s = lax.axis_index("s")
        tid = c * 16 + s
        pltpu.sync_copy(z_hbm.at[pl.ds(s * rpt, rpt)],
                        acc.at[pl.ds(s * rpt, rpt)])
        plsc.subcore_barrier()

        def window(wi, _):
            base = tid * ept + wi * w
            pltpu.sync_copy(ei_hbm.at[0].at[pl.ds(base, w)], idx_s)
            pltpu.sync_copy(ei_hbm.at[1].at[pl.ds(base, w)], idx_d)
            pltpu.sync_copy(wgt_hbm.at[pl.ds(base, w)], wbuf)
            pltpu.async_copy(hs_hbm.at[idx_s], rows, sem).wait()

            @plsc.parallel_loop(0, w, unroll=8)
            def edge(j):
                io = _io16()
                b = plsc.load_gather(wbuf, [io * 0 + j])
                rows[j, :] = rows[j, :] * b
            pltpu.sync_copy(rows, acc.at[idx_d], add=True)
            return 0

        lax.fori_loop(0, nwin, window, 0, unroll=False)
        plsc.subcore_barrier()
        pltpu.sync_copy(acc.at[pl.ds(s * rpt, rpt)],
                        out_hbm.at[c, pl.ds(s * rpt, rpt)])

    return k


def _att_mat(a_s, a_d):
    eye = jnp.eye(8, dtype=jnp.float32)
    As = (a_s[:, :, None] * eye[:, None, :]).reshape(128, 8)
    Ad = (a_d[:, :, None] * eye[:, None, :]).reshape(128, 8)
    return jnp.concatenate([As, Ad[:, ::-1], As, Ad], axis=1)


def _gat_layer(x, f_in, Wm, a_s, a_d, b, edge_index, z16, z32):
    ht, aa, aap, gm = _tc_dense(N, f_in, R_BLK)(x, Wm, _att_mat(a_s, a_d))
    gvec = jnp.concatenate([gm[0], jnp.zeros((8,), jnp.float32)])
    ext, sp = _sc_sweep1(N, E, SC_W1)(aa, gvec, edge_index, z16)
    up = _sc_sweep2(N, E, SC_W2)(ht, edge_index, ext, z32)
    xn = _tc_combine(N, R_BLK)(up, sp, aap, gm, ht, b.reshape(1, 128))
    return xn


def kernel(x, edge_index, edge_weight, W1, att_src1, att_dst1, b1,
           W2, att_src2, att_dst2, b2, Wg, bg, Wf, bf):
    z16 = jnp.zeros((N, 16), jnp.float32)
    z32 = jnp.zeros((N, 32), jnp.float32)

    degp = _sc_deg(N, E, SC_W1)(edge_index, edge_weight, z16)
    x2 = _gat_layer(x, 64, W1, att_src1, att_dst1, b1, edge_index, z16, z32)
    x3 = _gat_layer(x2, 128, W2, att_src2, att_dst2, b2, edge_index, z16, z32)

    hs, hg, dinv = _tc_gcn_pre(N, R_BLK)(x3, Wg, degp)
    accp = _sc_sweep3(N, E, SC_W1)(hs, edge_index, edge_weight, z16)
    y = _tc_head(N, R_BLK)(accp, hg, dinv, bg.reshape(1, 16),
                           Wf.reshape(1, 16), bf.reshape(1, 1))
    return y


# speedup vs baseline: 1.0692x; 1.0692x over previous
import functools

import jax
import jax.numpy as jnp
from jax import lax
from jax.experimental import pallas as pl
from jax.experimental.pallas import tpu as pltpu
import jax.experimental.pallas.tpu_sc as plsc

N = 50000
E = 800000
H = 8
C = 16
NT = 32
SC_W1 = 1000
SC_W2 = 200
R_BLK = 2000

_SC_PARAMS = pltpu.CompilerParams(
    use_tc_tiling_on_sc=False, needs_layout_passes=False)


def _io16():
    return lax.iota(jnp.int32, 16)


def _leaky(v):
    return jnp.where(v >= 0.0, v, 0.2 * v)


def _elu(v):
    return jnp.where(v > 0.0, v, jnp.exp(jnp.minimum(v, 0.0)) - 1.0)


def _tc_dense(n, f_in, r):
    nblk = n // r

    def body(x_ref, w_ref, ab_ref, ht_ref, aa_ref, aap_ref, gmax_ref):
        i = pl.program_id(0)
        h = jnp.dot(x_ref[...], w_ref[...], preferred_element_type=jnp.float32)
        for g in range(4):
            ht_ref[g] = h[:, 32 * g:32 * g + 32]
        aa_all = jnp.dot(h, ab_ref[...], preferred_element_type=jnp.float32)
        aa_ref[...] = aa_all[:, :16]
        aap_ref[...] = aa_all[:, 16:]
        bm = jnp.max(aa_all[:, :8], axis=0, keepdims=True)

        @pl.when(i == 0)
        def _():
            gmax_ref[...] = jnp.full((1, 8), -1e30, jnp.float32)

        gmax_ref[...] = jnp.maximum(gmax_ref[...], bm)

    return pl.pallas_call(
        body,
        grid=(nblk,),
        in_specs=[
            pl.BlockSpec((r, f_in), lambda i: (i, 0)),
            pl.BlockSpec((f_in, 128), lambda i: (0, 0)),
            pl.BlockSpec((128, 32), lambda i: (0, 0)),
        ],
        out_specs=[
            pl.BlockSpec((4, r, 32), lambda i: (0, i, 0)),
            pl.BlockSpec((r, 16), lambda i: (i, 0)),
            pl.BlockSpec((r, 16), lambda i: (i, 0)),
            pl.BlockSpec((1, 8), lambda i: (0, 0)),
        ],
        out_shape=[
            jax.ShapeDtypeStruct((4, n, 32), jnp.float32),
            jax.ShapeDtypeStruct((n, 16), jnp.float32),
            jax.ShapeDtypeStruct((n, 16), jnp.float32),
            jax.ShapeDtypeStruct((1, 8), jnp.float32),
        ],
    )


def _tc_combine(n, r):
    nblk = n // r

    def body(up_ref, sp_ref, aap_ref, gm_ref, ht_ref, b_ref, x_ref):
        as_ = aap_ref[:, :8]
        ad = aap_ref[:, 8:]
        g = gm_ref[...]
        eself = jnp.exp(_leaky(as_ + ad) - _leaky(ad + g))
        s_tot = sp_ref[0, :, :8] + sp_ref[1, :, :8] + eself
        rin = 1.0 / (s_tot + 1e-16)
        b = b_ref[...]
        for gi in range(4):
            es = jnp.concatenate(
                [jnp.broadcast_to(eself[:, 2 * gi:2 * gi + 1], (r, 16)),
                 jnp.broadcast_to(eself[:, 2 * gi + 1:2 * gi + 2], (r, 16))],
                axis=1)
            ri = jnp.concatenate(
                [jnp.broadcast_to(rin[:, 2 * gi:2 * gi + 1], (r, 16)),
                 jnp.broadcast_to(rin[:, 2 * gi + 1:2 * gi + 2], (r, 16))],
                axis=1)
            u = up_ref[gi, 0] + up_ref[gi, 1] + ht_ref[gi] * es
            x_ref[:, 32 * gi:32 * gi + 32] = _elu(
                u * ri + b[:, 32 * gi:32 * gi + 32])

    return pl.pallas_call(
        body,
        grid=(nblk,),
        in_specs=[
            pl.BlockSpec((4, 2, r, 32), lambda i: (0, 0, i, 0)),
            pl.BlockSpec((2, r, 16), lambda i: (0, i, 0)),
            pl.BlockSpec((r, 16), lambda i: (i, 0)),
            pl.BlockSpec((1, 8), lambda i: (0, 0)),
            pl.BlockSpec((4, r, 32), lambda i: (0, i, 0)),
            pl.BlockSpec((1, 128), lambda i: (0, 0)),
        ],
        out_specs=pl.BlockSpec((r, 128), lambda i: (i, 0)),
        out_shape=jax.ShapeDtypeStruct((n, 128), jnp.float32),
    )


def _x_block(up_ref, sp_ref, aap_ref, gm_ref, ht_ref, b_ref, r):
    as_ = aap_ref[:, :8]
    ad = aap_ref[:, 8:]
    g = gm_ref[...]
    eself = jnp.exp(_leaky(as_ + ad) - _leaky(ad + g))
    s_tot = sp_ref[0, :, :8] + sp_ref[1, :, :8] + eself
    rin = 1.0 / (s_tot + 1e-16)
    b = b_ref[...]
    pieces = []
    for gi in range(4):
        es = jnp.concatenate(
            [jnp.broadcast_to(eself[:, 2 * gi:2 * gi + 1], (r, 16)),
             jnp.broadcast_to(eself[:, 2 * gi + 1:2 * gi + 2], (r, 16))],
            axis=1)
        ri = jnp.concatenate(
            [jnp.broadcast_to(rin[:, 2 * gi:2 * gi + 1], (r, 16)),
             jnp.broadcast_to(rin[:, 2 * gi + 1:2 * gi + 2], (r, 16))],
            axis=1)
        u = up_ref[gi, 0] + up_ref[gi, 1] + ht_ref[gi] * es
        pieces.append(_elu(u * ri + b[:, 32 * gi:32 * gi + 32]))
    return jnp.concatenate(pieces, axis=1)


def _tc_combine_dense(n, r):
    nblk = n // r

    def body(up_ref, sp_ref, aap_ref, gm_ref, ht_ref, b_ref, w_ref, ab_ref,
             ht2_ref, aa_ref, aap2_ref, gmax_ref):
        i = pl.program_id(0)
        x = _x_block(up_ref, sp_ref, aap_ref, gm_ref, ht_ref, b_ref, r)
        h = jnp.dot(x, w_ref[...], preferred_element_type=jnp.float32)
        for g in range(4):
            ht2_ref[g] = h[:, 32 * g:32 * g + 32]
        aa_all = jnp.dot(h, ab_ref[...], preferred_element_type=jnp.float32)
        aa_ref[...] = aa_all[:, :16]
        aap2_ref[...] = aa_all[:, 16:]
        bm = jnp.max(aa_all[:, :8], axis=0, keepdims=True)

        @pl.when(i == 0)
        def _():
            gmax_ref[...] = jnp.full((1, 8), -1e30, jnp.float32)

        gmax_ref[...] = jnp.maximum(gmax_ref[...], bm)

    return pl.pallas_call(
        body,
        grid=(nblk,),
        in_specs=[
            pl.BlockSpec((4, 2, r, 32), lambda i: (0, 0, i, 0)),
            pl.BlockSpec((2, r, 16), lambda i: (0, i, 0)),
            pl.BlockSpec((r, 16), lambda i: (i, 0)),
            pl.BlockSpec((1, 8), lambda i: (0, 0)),
            pl.BlockSpec((4, r, 32), lambda i: (0, i, 0)),
            pl.BlockSpec((1, 128), lambda i: (0, 0)),
            pl.BlockSpec((128, 128), lambda i: (0, 0)),
            pl.BlockSpec((128, 32), lambda i: (0, 0)),
        ],
        out_specs=[
            pl.BlockSpec((4, r, 32), lambda i: (0, i, 0)),
            pl.BlockSpec((r, 16), lambda i: (i, 0)),
            pl.BlockSpec((r, 16), lambda i: (i, 0)),
            pl.BlockSpec((1, 8), lambda i: (0, 0)),
        ],
        out_shape=[
            jax.ShapeDtypeStruct((4, n, 32), jnp.float32),
            jax.ShapeDtypeStruct((n, 16), jnp.float32),
            jax.ShapeDtypeStruct((n, 16), jnp.float32),
            jax.ShapeDtypeStruct((1, 8), jnp.float32),
        ],
    )


def _tc_combine_gcn(n, r):
    nblk = n // r

    def body(up_ref, sp_ref, aap_ref, gm_ref, ht_ref, b_ref, wg_ref,
             degp_ref, hs_ref, hg_ref, dinv_ref):
        x = _x_block(up_ref, sp_ref, aap_ref, gm_ref, ht_ref, b_ref, r)
        hg = jnp.dot(x, wg_ref[...], preferred_element_type=jnp.float32)
        deg = degp_ref[0] + degp_ref[1] + 1.0
        dinv = lax.rsqrt(deg)
        hg_ref[...] = hg
        dinv_ref[...] = dinv
        hs_ref[...] = hg * dinv

    return pl.pallas_call(
        body,
        grid=(nblk,),
        in_specs=[
            pl.BlockSpec((4, 2, r, 32), lambda i: (0, 0, i, 0)),
            pl.BlockSpec((2, r, 16), lambda i: (0, i, 0)),
            pl.BlockSpec((r, 16), lambda i: (i, 0)),
            pl.BlockSpec((1, 8), lambda i: (0, 0)),
            pl.BlockSpec((4, r, 32), lambda i: (0, i, 0)),
            pl.BlockSpec((1, 128), lambda i: (0, 0)),
            pl.BlockSpec((128, 16), lambda i: (0, 0)),
            pl.BlockSpec((2, r, 16), lambda i: (0, i, 0)),
        ],
        out_specs=[
            pl.BlockSpec((r, 16), lambda i: (i, 0)),
            pl.BlockSpec((r, 16), lambda i: (i, 0)),
            pl.BlockSpec((r, 16), lambda i: (i, 0)),
        ],
        out_shape=[
            jax.ShapeDtypeStruct((n, 16), jnp.float32),
            jax.ShapeDtypeStruct((n, 16), jnp.float32),
            jax.ShapeDtypeStruct((n, 16), jnp.float32),
        ],
    )


def _tc_gcn_pre(n, r):
    nblk = n // r

    def body(x_ref, wg_ref, degp_ref, hs_ref, hg_ref, dinv_ref):
        hg = jnp.dot(x_ref[...], wg_ref[...], preferred_element_type=jnp.float32)
        deg = degp_ref[0] + degp_ref[1] + 1.0
        dinv = lax.rsqrt(deg)
        hg_ref[...] = hg
        dinv_ref[...] = dinv
        hs_ref[...] = hg * dinv

    return pl.pallas_call(
        body,
        grid=(nblk,),
        in_specs=[
            pl.BlockSpec((r, 128), lambda i: (i, 0)),
            pl.BlockSpec((128, 16), lambda i: (0, 0)),
            pl.BlockSpec((2, r, 16), lambda i: (0, i, 0)),
        ],
        out_specs=[
            pl.BlockSpec((r, 16), lambda i: (i, 0)),
            pl.BlockSpec((r, 16), lambda i: (i, 0)),
            pl.BlockSpec((r, 16), lambda i: (i, 0)),
        ],
        out_shape=[
            jax.ShapeDtypeStruct((n, 16), jnp.float32),
            jax.ShapeDtypeStruct((n, 16), jnp.float32),
            jax.ShapeDtypeStruct((n, 16), jnp.float32),
        ],
    )


def _tc_head(n, r):
    nblk = n // r

    def body(accp_ref, hg_ref, dinv_ref, bg_ref, wf_ref, bf_ref, y_ref):
        dinv = dinv_ref[...]
        acc = accp_ref[0] + accp_ref[1]
        out3 = _elu(dinv * acc + hg_ref[...] * dinv * dinv + bg_ref[...])
        y = jnp.sum(out3 * wf_ref[...], axis=1, keepdims=True) + bf_ref[...]
        y_ref[...] = jax.nn.sigmoid(y)

    return pl.pallas_call(
        body,
        grid=(nblk,),
        in_specs=[
            pl.BlockSpec((2, r, 16), lambda i: (0, i, 0)),
            pl.BlockSpec((r, 16), lambda i: (i, 0)),
            pl.BlockSpec((r, 16), lambda i: (i, 0)),
            pl.BlockSpec((1, 16), lambda i: (0, 0)),
            pl.BlockSpec((1, 16), lambda i: (0, 0)),
            pl.BlockSpec((1, 1), lambda i: (0, 0)),
        ],
        out_specs=pl.BlockSpec((r, 1), lambda i: (i, 0)),
        out_shape=jax.ShapeDtypeStruct((n, 1), jnp.float32),
    )


def _sc_sweep1(n, e, w):
    ept = e // NT
    nwin = ept // w
    rpt = n // 16
    mesh = plsc.VectorSubcoreMesh(core_axis_name="c", subcore_axis_name="s")

    @functools.partial(
        pl.kernel, mesh=mesh, compiler_params=_SC_PARAMS,
        out_type=[
            jax.ShapeDtypeStruct((8, e), jnp.float32),
            jax.ShapeDtypeStruct((2, n, 16), jnp.float32),
        ],
        scratch_types=[
            pltpu.VMEM((w,), jnp.int32),
            pltpu.VMEM((w,), jnp.int32),
            pltpu.VMEM((w, 16), jnp.float32),
            pltpu.VMEM((w, 16), jnp.float32),
            pltpu.VMEM((8 * w,), jnp.float32),
            pltpu.VMEM((w, 16), jnp.float32),
            pltpu.VMEM((16,), jnp.float32),
            pltpu.VMEM_SHARED((n, 16), jnp.float32),
            pltpu.SemaphoreType.DMA,
            pltpu.SemaphoreType.DMA,
        ])
    def k(aa_hbm, g_hbm, ei_hbm, z_hbm, ext_hbm, sp_hbm,
          idx_s, idx_d, rows_s, rows_d, exw, sw, gvec, s_acc, sem1, sem2):
        c = lax.axis_index("c")
        s = lax.axis_index("s")
        tid = c * 16 + s
        pltpu.sync_copy(z_hbm.at[pl.ds(s * rpt, rpt)],
                        s_acc.at[pl.ds(s * rpt, rpt)])
        pltpu.sync_copy(g_hbm, gvec)
        plsc.subcore_barrier()
        gv = gvec[...]

        def window(wi, _):
            base = tid * ept + wi * w
            pltpu.sync_copy(ei_hbm.at[0].at[pl.ds(base, w)], idx_s)
            pltpu.sync_copy(ei_hbm.at[1].at[pl.ds(base, w)], idx_d)
            cp1 = pltpu.async_copy(aa_hbm.at[idx_s], rows_s, sem1)
            cp2 = pltpu.async_copy(aa_hbm.at[idx_d], rows_d, sem2)
            cp1.wait()
            cp2.wait()

            @plsc.parallel_loop(0, w, unroll=8)
            def edge(j):
                io = _io16()
                va = rows_s[j, :]
                vd = rows_d[j, :]
                rot = lax.rev(vd, (0,))
                ex = jnp.exp(_leaky(va + rot) - _leaky(rot + gv))
                plsc.store_scatter(exw, [(io & 7) * w + j], ex, mask=io < 8)
                sw[j, :] = ex
            for hd in range(8):
                pltpu.sync_copy(exw.at[pl.ds(hd * w, w)],
                                ext_hbm.at[hd, pl.ds(base, w)])
            pltpu.sync_copy(sw, s_acc.at[idx_d], add=True)
            return 0

        lax.fori_loop(0, nwin, window, 0, unroll=False)
        plsc.subcore_barrier()
        pltpu.sync_copy(s_acc.at[pl.ds(s * rpt, rpt)],
                        sp_hbm.at[c, pl.ds(s * rpt, rpt)])

    return k


def _sc_deg(n, e, w):
    ept = e // NT
    nwin = ept // w
    rpt = n // 16
    mesh = plsc.VectorSubcoreMesh(core_axis_name="c", subcore_axis_name="s")

    @functools.partial(
        pl.kernel, mesh=mesh, compiler_params=_SC_PARAMS,
        out_type=jax.ShapeDtypeStruct((2, n, 16), jnp.float32),
        scratch_types=[
            pltpu.VMEM((w,), jnp.int32),
            pltpu.VMEM((w,), jnp.float32),
            pltpu.VMEM((w, 16), jnp.float32),
            pltpu.VMEM_SHARED((n, 16), jnp.float32),
        ])
    def k(ei_hbm, wgt_hbm, z_hbm, degp_hbm, idx_d, wbuf, wrows, deg_acc):
        c = lax.axis_index("c")
        s = lax.axis_index("s")
        tid = c * 16 + s
        pltpu.sync_copy(z_hbm.at[pl.ds(s * rpt, rpt)],
                        deg_acc.at[pl.ds(s * rpt, rpt)])
        plsc.subcore_barrier()

        def window(wi, _):
            base = tid * ept + wi * w
            pltpu.sync_copy(ei_hbm.at[1].at[pl.ds(base, w)], idx_d)
            pltpu.sync_copy(wgt_hbm.at[pl.ds(base, w)], wbuf)

            @plsc.parallel_loop(0, w, unroll=8)
            def edge(j):
                io = _io16()
                wrows[j, :] = plsc.load_gather(wbuf, [io * 0 + j])
            pltpu.sync_copy(wrows, deg_acc.at[idx_d], add=True)
            return 0

        lax.fori_loop(0, nwin, window, 0, unroll=False)
        plsc.subcore_barrier()
        pltpu.sync_copy(deg_acc.at[pl.ds(s * rpt, rpt)],
                        degp_hbm.at[c, pl.ds(s * rpt, rpt)])

    return k


def _sweep_pipeline(nwin, issue_a, wait_a, issue_d, wait_d, issue_g,
                    wait_g, compute, issue_s, wait_s):
    issue_a(0, 0)
    issue_d(0, 0)
    issue_a(1, 1)
    wait_a(0)
    issue_g(0, 0)
    wait_g(0)
    compute(0, 0)
    wait_d(0)
    issue_s(0, 0)
    wait_a(1)
    issue_d(1, 1)
    issue_g(1, 1)
    issue_a(2, 0)

    def pair(m, _):
        for i in (0, 1):
            b = (1 + i) & 1
            bn = 1 - b
            k = 1 + 2 * m + i
            wait_g(b)
            compute(k, b)
            wait_d(b)
            issue_s(k, b)

            @pl.when(k + 1 < nwin)
            def _():
                wait_a(bn)
                wait_s(bn)
                issue_d(k + 1, bn)
                issue_g(k + 1, bn)

            @pl.when(k + 2 < nwin)
            def _():
                issue_a(k + 2, b)
        return 0

    lax.fori_loop(0, (nwin - 1) // 2, pair, 0, unroll=False)
    wait_s(0)
    wait_s(1)


def _sc_sweep2(n, e, w):
    ept = e // NT
    nwin = ept // w
    rpt = n // 16
    mesh = plsc.VectorSubcoreMesh(core_axis_name="c", subcore_axis_name="s")

    @functools.partial(
        pl.kernel, mesh=mesh, compiler_params=_SC_PARAMS,
        out_type=jax.ShapeDtypeStruct((4, 2, n, 32), jnp.float32),
        scratch_types=[
            pltpu.VMEM((2, w), jnp.int32),
            pltpu.VMEM((2, w), jnp.int32),
            pltpu.VMEM((2, w, 32), jnp.float32),
            pltpu.VMEM((2, 2, w), jnp.float32),
            pltpu.VMEM_SHARED((n, 32), jnp.float32),
            pltpu.SemaphoreType.DMA,
            pltpu.SemaphoreType.DMA,
            pltpu.SemaphoreType.DMA,
            pltpu.SemaphoreType.DMA,
            pltpu.SemaphoreType.DMA,
            pltpu.SemaphoreType.DMA,
            pltpu.SemaphoreType.DMA,
            pltpu.SemaphoreType.DMA,
        ])
    def k(ht_hbm, ei_hbm, ext_hbm, z_hbm, out_hbm,
          idx_s, idx_ds, rows, ex2, acc, sa0, sa1, sd0, sd1, sg0, sg1,
          ss0, ss1):
        c = lax.axis_index("c")
        s = lax.axis_index("s")
        tid = c * 16 + s
        t0 = tid * ept
        semA = (sa0, sa1)
        semD = (sd0, sd1)
        semG = (sg0, sg1)
        semS = (ss0, ss1)

        for g in range(4):
            pltpu.sync_copy(z_hbm.at[pl.ds(s * rpt, rpt)],
                            acc.at[pl.ds(s * rpt, rpt)])
            plsc.subcore_barrier()

            def issue_a(k_, b, g=g):
                base = t0 + k_ * w
                pltpu.async_copy(ei_hbm.at[0].at[pl.ds(base, w)],
                                 idx_s.at[b], semA[b])
                pltpu.async_copy(ext_hbm.at[pl.ds(2 * g, 2), pl.ds(base, w)],
                                 ex2.at[b], semA[b])

            def wait_a(b, g=g):
                pltpu.make_async_copy(ei_hbm.at[0].at[pl.ds(0, w)],
                                      idx_s.at[b], semA[b]).wait()
                pltpu.make_async_copy(
                    ext_hbm.at[pl.ds(2 * g, 2), pl.ds(0, w)], ex2.at[b],
                    semA[b]).wait()

            def issue_d(k_, b):
                base = t0 + k_ * w
                pltpu.async_copy(ei_hbm.at[1].at[pl.ds(base, w)],
                                 idx_ds.at[b], semD[b])

            def wait_d(b):
                pltpu.make_async_copy(ei_hbm.at[1].at[pl.ds(0, w)],
                                      idx_ds.at[b], semD[b]).wait()

            def issue_g(k_, b, g=g):
                pltpu.async_copy(ht_hbm.at[g].at[idx_s.at[b]], rows.at[b],
                                 semG[b])

            def wait_g(b, g=g):
                pltpu.make_async_copy(ht_hbm.at[g].at[idx_s.at[b]],
                                      rows.at[b], semG[b]).wait()

            def compute(k_, b):
                @plsc.parallel_loop(0, w, unroll=8)
                def edge(j):
                    io = _io16()
                    b0 = plsc.load_gather(ex2.at[b, 0], [io * 0 + j])
                    b1 = plsc.load_gather(ex2.at[b, 1], [io * 0 + j])
                    rows[b, j, pl.ds(0, 16)] = rows[b, j, pl.ds(0, 16)] * b0
                    rows[b, j, pl.ds(16, 16)] = rows[b, j, pl.ds(16, 16)] * b1

            def issue_s(k_, b):
                pltpu.async_copy(rows.at[b], acc.at[idx_ds.at[b]], semS[b],
                                 add=True)

            def wait_s(b):
                pltpu.make_async_copy(rows.at[b], acc.at[idx_ds.at[b]],
                                      semS[b]).wait()

            _sweep_pipeline(nwin, issue_a, wait_a, issue_d, wait_d, issue_g,
                            wait_g, compute, issue_s, wait_s)
            plsc.subcore_barrier()
            pltpu.sync_copy(acc.at[pl.ds(s * rpt, rpt)],
                            out_hbm.at[g, c, pl.ds(s * rpt, rpt)])
            plsc.subcore_barrier()

    return k


def _sc_sweep3(n, e, w):
    ept = e // NT
    nwin = ept // w
    rpt = n // 16
    mesh = plsc.VectorSubcoreMesh(core_axis_name="c", subcore_axis_name="s")

    @functools.partial(
        pl.kernel, mesh=mesh, compiler_params=_SC_PARAMS,
        out_type=jax.ShapeDtypeStruct((2, n, 16), jnp.float32),
        scratch_types=[
            pltpu.VMEM((w,), jnp.int32),
            pltpu.VMEM((w,), jnp.int32),
            pltpu.VMEM((w,), jnp.float32),
            pltpu.VMEM((w, 16), jnp.float32),
            pltpu.VMEM_SHARED((n, 16), jnp.float32),
            pltpu.SemaphoreType.DMA,
        ])
    def k(hs_hbm, ei_hbm, wgt_hbm, z_hbm, out_hbm,
          idx_s, idx_d, wbuf, rows, acc, sem):
        c = lax.axis_index("c")
        s = lax.axis_index("s")
        tid = c * 16 + s
        pltpu.sync_copy(z_hbm.at[pl.ds(s * rpt, rpt)],
                        acc.at[pl.ds(s * rpt, rpt)])
        plsc.subcore_barrier()

        def window(wi, _):
            base = tid * ept + wi * w
            pltpu.sync_copy(ei_hbm.at[0].at[pl.ds(base, w)], idx_s)
            pltpu.sync_copy(ei_hbm.at[1].at[pl.ds(base, w)], idx_d)
            pltpu.sync_copy(wgt_hbm.at[pl.ds(base, w)], wbuf)
            pltpu.async_copy(hs_hbm.at[idx_s], rows, sem).wait()

            @plsc.parallel_loop(0, w, unroll=8)
            def edge(j):
                io = _io16()
                b = plsc.load_gather(wbuf, [io * 0 + j])
                rows[j, :] = rows[j, :] * b
            pltpu.sync_copy(rows, acc.at[idx_d], add=True)
            return 0

        lax.fori_loop(0, nwin, window, 0, unroll=False)
        plsc.subcore_barrier()
        pltpu.sync_copy(acc.at[pl.ds(s * rpt, rpt)],
                        out_hbm.at[c, pl.ds(s * rpt, rpt)])

    return k


def _att_mat(a_s, a_d):
    eye = jnp.eye(8, dtype=jnp.float32)
    As = (a_s[:, :, None] * eye[:, None, :]).reshape(128, 8)
    Ad = (a_d[:, :, None] * eye[:, None, :]).reshape(128, 8)
    return jnp.concatenate([As, Ad[:, ::-1], As, Ad], axis=1)


def _edge_sweeps(aa, gm, ht, edge_index, z16, z32):
    gvec = jnp.concatenate([gm[0], jnp.zeros((8,), jnp.float32)])
    ext, sp = _sc_sweep1(N, E, SC_W1)(aa, gvec, edge_index, z16)
    up = _sc_sweep2(N, E, SC_W2)(ht, edge_index, ext, z32)
    return up, sp


def kernel(x, edge_index, edge_weight, W1, att_src1, att_dst1, b1,
           W2, att_src2, att_dst2, b2, Wg, bg, Wf, bf):
    z16 = jnp.zeros((N, 16), jnp.float32)
    z32 = jnp.zeros((N, 32), jnp.float32)

    degp = _sc_deg(N, E, SC_W1)(edge_index, edge_weight, z16)
    ht1, aa1, aap1, gm1 = _tc_dense(N, 64, R_BLK)(
        x, W1, _att_mat(att_src1, att_dst1))
    up1, sp1 = _edge_sweeps(aa1, gm1, ht1, edge_index, z16, z32)
    ht2, aa2, aap2, gm2 = _tc_combine_dense(N, R_BLK)(
        up1, sp1, aap1, gm1, ht1, b1.reshape(1, 128), W2,
        _att_mat(att_src2, att_dst2))
    up2, sp2 = _edge_sweeps(aa2, gm2, ht2, edge_index, z16, z32)
    hs, hg, dinv = _tc_combine_gcn(N, R_BLK)(
        up2, sp2, aap2, gm2, ht2, b2.reshape(1, 128), Wg, degp)
    accp = _sc_sweep3(N, E, SC_W1)(hs, edge_index, edge_weight, z16)
    y = _tc_head(N, R_BLK)(accp, hg, dinv, bg.reshape(1, 16),
                           Wf.reshape(1, 16), bf.reshape(1, 1))
    return y


# speedup vs baseline: 1.1028x; 1.0313x over previous
import functools

import jax
import jax.numpy as jnp
from jax import lax
from jax.experimental import pallas as pl
from jax.experimental.pallas import tpu as pltpu
import jax.experimental.pallas.tpu_sc as plsc

N = 50000
E = 800000
H = 8
C = 16
NT = 32
SC_W1 = 1000
SC_W2 = 200
R_BLK = 2000

_SC_PARAMS = pltpu.CompilerParams(
    use_tc_tiling_on_sc=False, needs_layout_passes=False)


def _io16():
    return lax.iota(jnp.int32, 16)


def _leaky(v):
    return jnp.where(v >= 0.0, v, 0.2 * v)


def _elu(v):
    return jnp.where(v > 0.0, v, jnp.exp(jnp.minimum(v, 0.0)) - 1.0)


def _tc_dense(n, f_in, r):
    nblk = n // r

    def body(x_ref, w_ref, ab_ref, ht_ref, aa_ref, aap_ref, gmax_ref):
        i = pl.program_id(0)
        h = jnp.dot(x_ref[...], w_ref[...], preferred_element_type=jnp.float32)
        for g in range(4):
            ht_ref[g] = h[:, 32 * g:32 * g + 32]
        aa_all = jnp.dot(h, ab_ref[...], preferred_element_type=jnp.float32)
        aa_ref[...] = aa_all[:, :16]
        aap_ref[...] = aa_all[:, 16:]
        bm = jnp.max(aa_all[:, :8], axis=0, keepdims=True)

        @pl.when(i == 0)
        def _():
            gmax_ref[...] = jnp.full((1, 8), -1e30, jnp.float32)

        gmax_ref[...] = jnp.maximum(gmax_ref[...], bm)

    return pl.pallas_call(
        body,
        grid=(nblk,),
        in_specs=[
            pl.BlockSpec((r, f_in), lambda i: (i, 0)),
            pl.BlockSpec((f_in, 128), lambda i: (0, 0)),
            pl.BlockSpec((128, 32), lambda i: (0, 0)),
        ],
        out_specs=[
            pl.BlockSpec((4, r, 32), lambda i: (0, i, 0)),
            pl.BlockSpec((r, 16), lambda i: (i, 0)),
            pl.BlockSpec((r, 16), lambda i: (i, 0)),
            pl.BlockSpec((1, 8), lambda i: (0, 0)),
        ],
        out_shape=[
            jax.ShapeDtypeStruct((4, n, 32), jnp.float32),
            jax.ShapeDtypeStruct((n, 16), jnp.float32),
            jax.ShapeDtypeStruct((n, 16), jnp.float32),
            jax.ShapeDtypeStruct((1, 8), jnp.float32),
        ],
    )


def _tc_combine(n, r):
    nblk = n // r

    def body(up_ref, sp_ref, aap_ref, gm_ref, ht_ref, b_ref, x_ref):
        as_ = aap_ref[:, :8]
        ad = aap_ref[:, 8:]
        g = gm_ref[...]
        eself = jnp.exp(_leaky(as_ + ad) - _leaky(ad + g))
        s_tot = sp_ref[0, :, :8] + sp_ref[1, :, :8] + eself
        rin = 1.0 / (s_tot + 1e-16)
        b = b_ref[...]
        for gi in range(4):
            es = jnp.concatenate(
                [jnp.broadcast_to(eself[:, 2 * gi:2 * gi + 1], (r, 16)),
                 jnp.broadcast_to(eself[:, 2 * gi + 1:2 * gi + 2], (r, 16))],
                axis=1)
            ri = jnp.concatenate(
                [jnp.broadcast_to(rin[:, 2 * gi:2 * gi + 1], (r, 16)),
                 jnp.broadcast_to(rin[:, 2 * gi + 1:2 * gi + 2], (r, 16))],
                axis=1)
            u = up_ref[gi, 0] + up_ref[gi, 1] + ht_ref[gi] * es
            x_ref[:, 32 * gi:32 * gi + 32] = _elu(
                u * ri + b[:, 32 * gi:32 * gi + 32])

    return pl.pallas_call(
        body,
        grid=(nblk,),
        in_specs=[
            pl.BlockSpec((4, 2, r, 32), lambda i: (0, 0, i, 0)),
            pl.BlockSpec((2, r, 16), lambda i: (0, i, 0)),
            pl.BlockSpec((r, 16), lambda i: (i, 0)),
            pl.BlockSpec((1, 8), lambda i: (0, 0)),
            pl.BlockSpec((4, r, 32), lambda i: (0, i, 0)),
            pl.BlockSpec((1, 128), lambda i: (0, 0)),
        ],
        out_specs=pl.BlockSpec((r, 128), lambda i: (i, 0)),
        out_shape=jax.ShapeDtypeStruct((n, 128), jnp.float32),
    )


def _x_block(up_ref, sp_ref, aap_ref, gm_ref, ht_ref, b_ref, r):
    as_ = aap_ref[:, :8]
    ad = aap_ref[:, 8:]
    g = gm_ref[...]
    eself = jnp.exp(_leaky(as_ + ad) - _leaky(ad + g))
    s_tot = sp_ref[0, :, :8] + sp_ref[1, :, :8] + eself
    rin = 1.0 / (s_tot + 1e-16)
    b = b_ref[...]
    pieces = []
    for gi in range(4):
        es = jnp.concatenate(
            [jnp.broadcast_to(eself[:, 2 * gi:2 * gi + 1], (r, 16)),
             jnp.broadcast_to(eself[:, 2 * gi + 1:2 * gi + 2], (r, 16))],
            axis=1)
        ri = jnp.concatenate(
            [jnp.broadcast_to(rin[:, 2 * gi:2 * gi + 1], (r, 16)),
             jnp.broadcast_to(rin[:, 2 * gi + 1:2 * gi + 2], (r, 16))],
            axis=1)
        u = up_ref[gi, 0] + up_ref[gi, 1] + ht_ref[gi] * es
        pieces.append(_elu(u * ri + b[:, 32 * gi:32 * gi + 32]))
    return jnp.concatenate(pieces, axis=1)


def _tc_combine_dense(n, r):
    nblk = n // r

    def body(up_ref, sp_ref, aap_ref, gm_ref, ht_ref, b_ref, w_ref, ab_ref,
             ht2_ref, aa_ref, aap2_ref, gmax_ref):
        i = pl.program_id(0)
        x = _x_block(up_ref, sp_ref, aap_ref, gm_ref, ht_ref, b_ref, r)
        h = jnp.dot(x, w_ref[...], preferred_element_type=jnp.float32)
        for g in range(4):
            ht2_ref[g] = h[:, 32 * g:32 * g + 32]
        aa_all = jnp.dot(h, ab_ref[...], preferred_element_type=jnp.float32)
        aa_ref[...] = aa_all[:, :16]
        aap2_ref[...] = aa_all[:, 16:]
        bm = jnp.max(aa_all[:, :8], axis=0, keepdims=True)

        @pl.when(i == 0)
        def _():
            gmax_ref[...] = jnp.full((1, 8), -1e30, jnp.float32)

        gmax_ref[...] = jnp.maximum(gmax_ref[...], bm)

    return pl.pallas_call(
        body,
        grid=(nblk,),
        in_specs=[
            pl.BlockSpec((4, 2, r, 32), lambda i: (0, 0, i, 0)),
            pl.BlockSpec((2, r, 16), lambda i: (0, i, 0)),
            pl.BlockSpec((r, 16), lambda i: (i, 0)),
            pl.BlockSpec((1, 8), lambda i: (0, 0)),
            pl.BlockSpec((4, r, 32), lambda i: (0, i, 0)),
            pl.BlockSpec((1, 128), lambda i: (0, 0)),
            pl.BlockSpec((128, 128), lambda i: (0, 0)),
            pl.BlockSpec((128, 32), lambda i: (0, 0)),
        ],
        out_specs=[
            pl.BlockSpec((4, r, 32), lambda i: (0, i, 0)),
            pl.BlockSpec((r, 16), lambda i: (i, 0)),
            pl.BlockSpec((r, 16), lambda i: (i, 0)),
            pl.BlockSpec((1, 8), lambda i: (0, 0)),
        ],
        out_shape=[
            jax.ShapeDtypeStruct((4, n, 32), jnp.float32),
            jax.ShapeDtypeStruct((n, 16), jnp.float32),
            jax.ShapeDtypeStruct((n, 16), jnp.float32),
            jax.ShapeDtypeStruct((1, 8), jnp.float32),
        ],
    )


def _tc_combine_gcn(n, r):
    nblk = n // r

    def body(up_ref, sp_ref, aap_ref, gm_ref, ht_ref, b_ref, wg_ref,
             degp_ref, hs_ref, hg_ref, dinv_ref):
        x = _x_block(up_ref, sp_ref, aap_ref, gm_ref, ht_ref, b_ref, r)
        hg = jnp.dot(x, wg_ref[...], preferred_element_type=jnp.float32)
        deg = degp_ref[0] + degp_ref[1] + 1.0
        dinv = lax.rsqrt(deg)
        hg_ref[...] = hg
        dinv_ref[...] = dinv
        hs_ref[...] = hg * dinv

    return pl.pallas_call(
        body,
        grid=(nblk,),
        in_specs=[
            pl.BlockSpec((4, 2, r, 32), lambda i: (0, 0, i, 0)),
            pl.BlockSpec((2, r, 16), lambda i: (0, i, 0)),
            pl.BlockSpec((r, 16), lambda i: (i, 0)),
            pl.BlockSpec((1, 8), lambda i: (0, 0)),
            pl.BlockSpec((4, r, 32), lambda i: (0, i, 0)),
            pl.BlockSpec((1, 128), lambda i: (0, 0)),
            pl.BlockSpec((128, 16), lambda i: (0, 0)),
            pl.BlockSpec((2, r, 16), lambda i: (0, i, 0)),
        ],
        out_specs=[
            pl.BlockSpec((r, 16), lambda i: (i, 0)),
            pl.BlockSpec((r, 16), lambda i: (i, 0)),
            pl.BlockSpec((r, 16), lambda i: (i, 0)),
        ],
        out_shape=[
            jax.ShapeDtypeStruct((n, 16), jnp.float32),
            jax.ShapeDtypeStruct((n, 16), jnp.float32),
            jax.ShapeDtypeStruct((n, 16), jnp.float32),
        ],
    )


def _tc_gcn_pre(n, r):
    nblk = n // r

    def body(x_ref, wg_ref, degp_ref, hs_ref, hg_ref, dinv_ref):
        hg = jnp.dot(x_ref[...], wg_ref[...], preferred_element_type=jnp.float32)
        deg = degp_ref[0] + degp_ref[1] + 1.0
        dinv = lax.rsqrt(deg)
        hg_ref[...] = hg
        dinv_ref[...] = dinv
        hs_ref[...] = hg * dinv

    return pl.pallas_call(
        body,
        grid=(nblk,),
        in_specs=[
            pl.BlockSpec((r, 128), lambda i: (i, 0)),
            pl.BlockSpec((128, 16), lambda i: (0, 0)),
            pl.BlockSpec((2, r, 16), lambda i: (0, i, 0)),
        ],
        out_specs=[
            pl.BlockSpec((r, 16), lambda i: (i, 0)),
            pl.BlockSpec((r, 16), lambda i: (i, 0)),
            pl.BlockSpec((r, 16), lambda i: (i, 0)),
        ],
        out_shape=[
            jax.ShapeDtypeStruct((n, 16), jnp.float32),
            jax.ShapeDtypeStruct((n, 16), jnp.float32),
            jax.ShapeDtypeStruct((n, 16), jnp.float32),
        ],
    )


def _tc_head(n, r):
    nblk = n // r

    def body(accp_ref, hg_ref, dinv_ref, bg_ref, wf_ref, bf_ref, y_ref):
        dinv = dinv_ref[...]
        acc = accp_ref[0] + accp_ref[1]
        out3 = _elu(dinv * acc + hg_ref[...] * dinv * dinv + bg_ref[...])
        y = jnp.sum(out3 * wf_ref[...], axis=1, keepdims=True) + bf_ref[...]
        y_ref[...] = jax.nn.sigmoid(y)

    return pl.pallas_call(
        body,
        grid=(nblk,),
        in_specs=[
            pl.BlockSpec((2, r, 16), lambda i: (0, i, 0)),
            pl.BlockSpec((r, 16), lambda i: (i, 0)),
            pl.BlockSpec((r, 16), lambda i: (i, 0)),
            pl.BlockSpec((1, 16), lambda i: (0, 0)),
            pl.BlockSpec((1, 16), lambda i: (0, 0)),
            pl.BlockSpec((1, 1), lambda i: (0, 0)),
        ],
        out_specs=pl.BlockSpec((r, 1), lambda i: (i, 0)),
        out_shape=jax.ShapeDtypeStruct((n, 1), jnp.float32),
    )


def _sc_sweep1(n, e, w):
    ept = e // NT
    nwin = ept // w
    rpt = n // 16
    mesh = plsc.VectorSubcoreMesh(core_axis_name="c", subcore_axis_name="s")

    @functools.partial(
        pl.kernel, mesh=mesh, compiler_params=_SC_PARAMS,
        out_type=[
            jax.ShapeDtypeStruct((8, e), jnp.float32),
            jax.ShapeDtypeStruct((2, n, 16), jnp.float32),
        ],
        scratch_types=[
            pltpu.VMEM((w,), jnp.int32),
            pltpu.VMEM((w,), jnp.int32),
            pltpu.VMEM((w, 16), jnp.float32),
            pltpu.VMEM((w, 16), jnp.float32),
            pltpu.VMEM((8 * w,), jnp.float32),
            pltpu.VMEM((w, 16), jnp.float32),
            pltpu.VMEM((16,), jnp.float32),
            pltpu.VMEM_SHARED((n, 16), jnp.float32),
            pltpu.SemaphoreType.DMA,
            pltpu.SemaphoreType.DMA,
        ])
    def k(aa_hbm, g_hbm, ei_hbm, z_hbm, ext_hbm, sp_hbm,
          idx_s, idx_d, rows_s, rows_d, exw, sw, gvec, s_acc, sem1, sem2):
        c = lax.axis_index("c")
        s = lax.axis_index("s")
        tid = c * 16 + s
        pltpu.sync_copy(z_hbm.at[pl.ds(s * rpt, rpt)],
                        s_acc.at[pl.ds(s * rpt, rpt)])
        pltpu.sync_copy(g_hbm, gvec)
        plsc.subcore_barrier()
        gv = gvec[...]

        def window(wi, _):
            base = tid * ept + wi * w
            ci1 = pltpu.async_copy(ei_hbm.at[0].at[pl.ds(base, w)], idx_s,
                                   sem1)
            ci2 = pltpu.async_copy(ei_hbm.at[1].at[pl.ds(base, w)], idx_d,
                                   sem2)
            ci1.wait()
            ci2.wait()
            cp1 = pltpu.async_copy(aa_hbm.at[idx_s], rows_s, sem1)
            cp2 = pltpu.async_copy(aa_hbm.at[idx_d], rows_d, sem2)
            cp1.wait()
            cp2.wait()

            @plsc.parallel_loop(0, w, unroll=8)
            def edge(j):
                io = _io16()
                va = rows_s[j, :]
                vd = rows_d[j, :]
                rot = lax.rev(vd, (0,))
                ex = jnp.exp(_leaky(va + rot) - _leaky(rot + gv))
                plsc.store_scatter(exw, [(io & 7) * w + j], ex, mask=io < 8)
                sw[j, :] = ex
            cps = [pltpu.async_copy(exw.at[pl.ds(hd * w, w)],
                                    ext_hbm.at[hd, pl.ds(base, w)], sem1)
                   for hd in range(8)]
            cpa = pltpu.async_copy(sw, s_acc.at[idx_d], sem2, add=True)
            for cp in cps:
                cp.wait()
            cpa.wait()
            return 0

        lax.fori_loop(0, nwin, window, 0, unroll=False)
        plsc.subcore_barrier()
        pltpu.sync_copy(s_acc.at[pl.ds(s * rpt, rpt)],
                        sp_hbm.at[c, pl.ds(s * rpt, rpt)])

    return k


def _sc_deg(n, e, w):
    ept = e // NT
    nwin = ept // w
    rpt = n // 16
    mesh = plsc.VectorSubcoreMesh(core_axis_name="c", subcore_axis_name="s")

    @functools.partial(
        pl.kernel, mesh=mesh, compiler_params=_SC_PARAMS,
        out_type=jax.ShapeDtypeStruct((2, n, 16), jnp.float32),
        scratch_types=[
            pltpu.VMEM((w,), jnp.int32),
            pltpu.VMEM((w,), jnp.float32),
            pltpu.VMEM((w, 16), jnp.float32),
            pltpu.VMEM_SHARED((n, 16), jnp.float32),
            pltpu.SemaphoreType.DMA,
        ])
    def k(ei_hbm, wgt_hbm, z_hbm, degp_hbm, idx_d, wbuf, wrows, deg_acc,
          sem):
        c = lax.axis_index("c")
        s = lax.axis_index("s")
        tid = c * 16 + s
        pltpu.sync_copy(z_hbm.at[pl.ds(s * rpt, rpt)],
                        deg_acc.at[pl.ds(s * rpt, rpt)])
        plsc.subcore_barrier()

        def window(wi, _):
            base = tid * ept + wi * w
            c1 = pltpu.async_copy(ei_hbm.at[1].at[pl.ds(base, w)], idx_d, sem)
            c2 = pltpu.async_copy(wgt_hbm.at[pl.ds(base, w)], wbuf, sem)
            c1.wait()
            c2.wait()

            @plsc.parallel_loop(0, w, unroll=8)
            def edge(j):
                io = _io16()
                wrows[j, :] = plsc.load_gather(wbuf, [io * 0 + j])
            pltpu.sync_copy(wrows, deg_acc.at[idx_d], add=True)
            return 0

        lax.fori_loop(0, nwin, window, 0, unroll=False)
        plsc.subcore_barrier()
        pltpu.sync_copy(deg_acc.at[pl.ds(s * rpt, rpt)],
                        degp_hbm.at[c, pl.ds(s * rpt, rpt)])

    return k


def _sweep_pipeline(nwin, issue_a, wait_a, issue_d, wait_d, issue_g,
                    wait_g, compute, issue_s, wait_s):
    issue_a(0, 0)
    issue_d(0, 0)
    issue_a(1, 1)
    wait_a(0)
    issue_g(0, 0)
    wait_g(0)
    compute(0, 0)
    wait_d(0)
    issue_s(0, 0)
    wait_a(1)
    issue_d(1, 1)
    issue_g(1, 1)
    issue_a(2, 0)

    def pair(m, _):
        for i in (0, 1):
            b = (1 + i) & 1
            bn = 1 - b
            k = 1 + 2 * m + i
            wait_g(b)
            compute(k, b)
            wait_d(b)
            issue_s(k, b)

            @pl.when(k + 1 < nwin)
            def _():
                wait_a(bn)
                wait_s(bn)
                issue_d(k + 1, bn)
                issue_g(k + 1, bn)

            @pl.when(k + 2 < nwin)
            def _():
                issue_a(k + 2, b)
        return 0

    lax.fori_loop(0, (nwin - 1) // 2, pair, 0, unroll=False)
    wait_s(0)
    wait_s(1)


def _sc_sweep2(n, e, w):
    ept = e // NT
    nwin = ept // w
    rpt = n // 16
    mesh = plsc.VectorSubcoreMesh(core_axis_name="c", subcore_axis_name="s")

    @functools.partial(
        pl.kernel, mesh=mesh, compiler_params=_SC_PARAMS,
        out_type=jax.ShapeDtypeStruct((4, 2, n, 32), jnp.float32),
        scratch_types=[
            pltpu.VMEM((2, w), jnp.int32),
            pltpu.VMEM((2, w), jnp.int32),
            pltpu.VMEM((2, w, 32), jnp.float32),
            pltpu.VMEM((2, 2, w), jnp.float32),
            pltpu.VMEM_SHARED((n, 32), jnp.float32),
            pltpu.SemaphoreType.DMA,
            pltpu.SemaphoreType.DMA,
            pltpu.SemaphoreType.DMA,
            pltpu.SemaphoreType.DMA,
            pltpu.SemaphoreType.DMA,
            pltpu.SemaphoreType.DMA,
            pltpu.SemaphoreType.DMA,
            pltpu.SemaphoreType.DMA,
        ])
    def k(ht_hbm, ei_hbm, ext_hbm, z_hbm, out_hbm,
          idx_s, idx_ds, rows, ex2, acc, sa0, sa1, sd0, sd1, sg0, sg1,
          ss0, ss1):
        c = lax.axis_index("c")
        s = lax.axis_index("s")
        tid = c * 16 + s
        t0 = tid * ept
        semA = (sa0, sa1)
        semD = (sd0, sd1)
        semG = (sg0, sg1)
        semS = (ss0, ss1)

        for g in range(4):
            pltpu.sync_copy(z_hbm.at[pl.ds(s * rpt, rpt)],
                            acc.at[pl.ds(s * rpt, rpt)])
            plsc.subcore_barrier()

            def issue_a(k_, b, g=g):
                base = t0 + k_ * w
                pltpu.async_copy(ei_hbm.at[0].at[pl.ds(base, w)],
                                 idx_s.at[b], semA[b])
                pltpu.async_copy(ext_hbm.at[pl.ds(2 * g, 2), pl.ds(base, w)],
                                 ex2.at[b], semA[b])

            def wait_a(b, g=g):
                pltpu.make_async_copy(ei_hbm.at[0].at[pl.ds(0, w)],
                                      idx_s.at[b], semA[b]).wait()
                pltpu.make_async_copy(
                    ext_hbm.at[pl.ds(2 * g, 2), pl.ds(0, w)], ex2.at[b],
                    semA[b]).wait()

            def issue_d(k_, b):
                base = t0 + k_ * w
                pltpu.async_copy(ei_hbm.at[1].at[pl.ds(base, w)],
                                 idx_ds.at[b], semD[b])

            def wait_d(b):
                pltpu.make_async_copy(ei_hbm.at[1].at[pl.ds(0, w)],
                                      idx_ds.at[b], semD[b]).wait()

            def issue_g(k_, b, g=g):
                pltpu.async_copy(ht_hbm.at[g].at[idx_s.at[b]], rows.at[b],
                                 semG[b])

            def wait_g(b, g=g):
                pltpu.make_async_copy(ht_hbm.at[g].at[idx_s.at[b]],
                                      rows.at[b], semG[b]).wait()

            def compute(k_, b):
                @plsc.parallel_loop(0, w, unroll=8)
                def edge(j):
                    io = _io16()
                    b0 = plsc.load_gather(ex2.at[b, 0], [io * 0 + j])
                    b1 = plsc.load_gather(ex2.at[b, 1], [io * 0 + j])
                    rows[b, j, pl.ds(0, 16)] = rows[b, j, pl.ds(0, 16)] * b0
                    rows[b, j, pl.ds(16, 16)] = rows[b, j, pl.ds(16, 16)] * b1

            def issue_s(k_, b):
                pltpu.async_copy(rows.at[b], acc.at[idx_ds.at[b]], semS[b],
                                 add=True)

            def wait_s(b):
                pltpu.make_async_copy(rows.at[b], acc.at[idx_ds.at[b]],
                                      semS[b]).wait()

            _sweep_pipeline(nwin, issue_a, wait_a, issue_d, wait_d, issue_g,
                            wait_g, compute, issue_s, wait_s)
            plsc.subcore_barrier()
            pltpu.sync_copy(acc.at[pl.ds(s * rpt, rpt)],
                            out_hbm.at[g, c, pl.ds(s * rpt, rpt)])
            plsc.subcore_barrier()

    return k


def _sc_sweep3(n, e, w):
    ept = e // NT
    nwin = ept // w
    rpt = n // 16
    mesh = plsc.VectorSubcoreMesh(core_axis_name="c", subcore_axis_name="s")

    @functools.partial(
        pl.kernel, mesh=mesh, compiler_params=_SC_PARAMS,
        out_type=jax.ShapeDtypeStruct((2, n, 16), jnp.float32),
        scratch_types=[
            pltpu.VMEM((w,), jnp.int32),
            pltpu.VMEM((w,), jnp.int32),
            pltpu.VMEM((w,), jnp.float32),
            pltpu.VMEM((w, 16), jnp.float32),
            pltpu.VMEM_SHARED((n, 16), jnp.float32),
            pltpu.SemaphoreType.DMA,
        ])
    def k(hs_hbm, ei_hbm, wgt_hbm, z_hbm, out_hbm,
          idx_s, idx_d, wbuf, rows, acc, sem):
        c = lax.axis_index("c")
        s = lax.axis_index("s")
        tid = c * 16 + s
        pltpu.sync_copy(z_hbm.at[pl.ds(s * rpt, rpt)],
                        acc.at[pl.ds(s * rpt, rpt)])
        plsc.subcore_barrier()

        def window(wi, _):
            base = tid * ept + wi * w
            c1 = pltpu.async_copy(ei_hbm.at[0].at[pl.ds(base, w)], idx_s, sem)
            c2 = pltpu.async_copy(ei_hbm.at[1].at[pl.ds(base, w)], idx_d, sem)
            c3 = pltpu.async_copy(wgt_hbm.at[pl.ds(base, w)], wbuf, sem)
            c1.wait()
            c2.wait()
            c3.wait()
            pltpu.async_copy(hs_hbm.at[idx_s], rows, sem).wait()

            @plsc.parallel_loop(0, w, unroll=8)
            def edge(j):
                io = _io16()
                b = plsc.load_gather(wbuf, [io * 0 + j])
                rows[j, :] = rows[j, :] * b
            pltpu.sync_copy(rows, acc.at[idx_d], add=True)
            return 0

        lax.fori_loop(0, nwin, window, 0, unroll=False)
        plsc.subcore_barrier()
        pltpu.sync_copy(acc.at[pl.ds(s * rpt, rpt)],
                        out_hbm.at[c, pl.ds(s * rpt, rpt)])

    return k


def _att_mat(a_s, a_d):
    eye = jnp.eye(8, dtype=jnp.float32)
    As = (a_s[:, :, None] * eye[:, None, :]).reshape(128, 8)
    Ad = (a_d[:, :, None] * eye[:, None, :]).reshape(128, 8)
    return jnp.concatenate([As, Ad[:, ::-1], As, Ad], axis=1)


def _edge_sweeps(aa, gm, ht, edge_index, z16, z32):
    gvec = jnp.concatenate([gm[0], jnp.zeros((8,), jnp.float32)])
    ext, sp = _sc_sweep1(N, E, SC_W1)(aa, gvec, edge_index, z16)
    up = _sc_sweep2(N, E, SC_W2)(ht, edge_index, ext, z32)
    return up, sp


def kernel(x, edge_index, edge_weight, W1, att_src1, att_dst1, b1,
           W2, att_src2, att_dst2, b2, Wg, bg, Wf, bf):
    z16 = jnp.zeros((N, 16), jnp.float32)
    z32 = jnp.zeros((N, 32), jnp.float32)

    degp = _sc_deg(N, E, SC_W1)(edge_index, edge_weight, z16)
    ht1, aa1, aap1, gm1 = _tc_dense(N, 64, R_BLK)(
        x, W1, _att_mat(att_src1, att_dst1))
    up1, sp1 = _edge_sweeps(aa1, gm1, ht1, edge_index, z16, z32)
    ht2, aa2, aap2, gm2 = _tc_combine_dense(N, R_BLK)(
        up1, sp1, aap1, gm1, ht1, b1.reshape(1, 128), W2,
        _att_mat(att_src2, att_dst2))
    up2, sp2 = _edge_sweeps(aa2, gm2, ht2, edge_index, z16, z32)
    hs, hg, dinv = _tc_combine_gcn(N, R_BLK)(
        up2, sp2, aap2, gm2, ht2, b2.reshape(1, 128), Wg, degp)
    accp = _sc_sweep3(N, E, SC_W1)(hs, edge_index, edge_weight, z16)
    y = _tc_head(N, R_BLK)(accp, hg, dinv, bg.reshape(1, 16),
                           Wf.reshape(1, 16), bf.reshape(1, 1))
    return y
